# baseline (speedup 1.0000x reference)
import functools

import jax
import jax.numpy as jnp
from jax import lax
from jax.experimental import pallas as pl
from jax.experimental.pallas import tpu as pltpu, tpu_sc as plsc

EMBED = 64
B = 4096
L = 200

NC = 2
NS = 16
NW = NC * NS
BBLK = 128
NB = B // BBLK
NUNIT = (L * NB) // NW
EG = EMBED // 8
LANES = 16


def _emit_unit(u, rows_v, pos_v, stag_v, out_hbm, osem):
    l = u >> 5
    b128 = u & (NB - 1)
    iot = lax.iota(jnp.int32, LANES)
    rowvs = [iot + bb * LANES for bb in range(8)]
    prow = [pos_v[l, pl.ds(j * LANES, LANES)] for j in range(EMBED // LANES)]

    for e8 in range(EG):
        for er in range(8):
            e = e8 * 8 + er
            colv = jnp.full((LANES,), e, jnp.int32)
            posv = jnp.full((LANES,), prow[e // LANES][e % LANES], jnp.float32)
            vals = [plsc.load_gather(rows_v, [rowvs[bb], colv])
                    for bb in range(8)]
            for bb in range(8):
                stag_v[e8, er, pl.ds(bb * LANES, LANES)] = vals[bb] + posv
        pltpu.async_copy(stag_v.at[e8], out_hbm.at[l, e8, b128], osem)


def _drain_out(out_hbm, stag_v, osem, u):
    l = u >> 5
    b128 = u & (NB - 1)
    for e8 in range(EG):
        pltpu.make_async_copy(stag_v.at[e8], out_hbm.at[l, e8, b128], osem).wait()


def _body(xt_hbm, tok_hbm, pos_hbm, out_hbm, idx_v, rows0, rows1,
          stag0, stag1, pos_v, gsem0, gsem1, osem0, osem1):
    wid = lax.axis_index("s") * NC + lax.axis_index("c")
    ubase = wid * NUNIT
    pltpu.sync_copy(xt_hbm.at[pl.ds(ubase, NUNIT)], idx_v)
    pltpu.sync_copy(pos_hbm.at[pl.ds(0, L)], pos_v)

    pltpu.async_copy(tok_hbm.at[idx_v.at[0]], rows0, gsem0)

    def pair_body(i, carry):
        t = i * 2

        pltpu.async_copy(tok_hbm.at[idx_v.at[t + 1]], rows1, gsem1)
        pltpu.make_async_copy(tok_hbm.at[idx_v.at[t]], rows0, gsem0).wait()

        @pl.when(i > 0)
        def _():
            _drain_out(out_hbm, stag0, osem0, ubase + t - 2)

        _emit_unit(ubase + t, rows0, pos_v, stag0, out_hbm, osem0)

        @pl.when(i < NUNIT // 2 - 1)
        def _():
            pltpu.async_copy(tok_hbm.at[idx_v.at[t + 2]], rows0, gsem0)

        pltpu.make_async_copy(tok_hbm.at[idx_v.at[t + 1]], rows1, gsem1).wait()

        @pl.when(i > 0)
        def _():
            _drain_out(out_hbm, stag1, osem1, ubase + t - 1)

        _emit_unit(ubase + t + 1, rows1, pos_v, stag1, out_hbm, osem1)
        return carry

    lax.fori_loop(0, NUNIT // 2, pair_body, 0)
    _drain_out(out_hbm, stag0, osem0, ubase + NUNIT - 2)
    _drain_out(out_hbm, stag1, osem1, ubase + NUNIT - 1)


@jax.jit
def kernel(x, token_table, pos_table):
    xt = x.T.reshape(L * NB, BBLK)
    mesh = plsc.VectorSubcoreMesh(core_axis_name="c", subcore_axis_name="s")
    run = pl.kernel(
        _body,
        mesh=mesh,
        compiler_params=pltpu.CompilerParams(
            use_tc_tiling_on_sc=False, needs_layout_passes=False,
            disable_bounds_checks=True),
        out_type=jax.ShapeDtypeStruct((L, EG, NB, 8, BBLK), jnp.float32),
        scratch_types=[
            pltpu.VMEM((NUNIT, BBLK), jnp.int32),
            pltpu.VMEM((BBLK, EMBED), jnp.float32),
            pltpu.VMEM((BBLK, EMBED), jnp.float32),
            pltpu.VMEM((EG, 8, BBLK), jnp.float32),
            pltpu.VMEM((EG, 8, BBLK), jnp.float32),
            pltpu.VMEM((L, EMBED), jnp.float32),
            pltpu.SemaphoreType.DMA,
            pltpu.SemaphoreType.DMA,
            pltpu.SemaphoreType.DMA,
            pltpu.SemaphoreType.DMA,
        ],
    )
    out5 = run(xt, token_table, pos_table)
    return out5.transpose(2, 4, 0, 1, 3).reshape(B, L, EMBED)


# speedup vs baseline: 1.2515x; 1.2515x over previous
import functools

import jax
import jax.numpy as jnp
from jax import lax
from jax.experimental import pallas as pl
from jax.experimental.pallas import tpu as pltpu, tpu_sc as plsc

EMBED = 64
B = 4096
L = 200

NC = 2
NS = 16
NW = NC * NS
BBLK = 128
NB = B // BBLK
NUNIT = (L * NB) // NW
EG = EMBED // 8
LANES = 16
ROWPAD = 72


def _emit_unit(u, rows_v, pad_v, pos_v, stag_v, out_hbm, osem):
    l = u >> 5
    b128 = u & (NB - 1)
    iot = lax.iota(jnp.int32, LANES)
    rowvs = [iot + bb * LANES for bb in range(8)]
    prow = [pos_v[l, pl.ds(j * LANES, LANES)] for j in range(EMBED // LANES)]

    def cp_body(k, carry):
        r = k * 4
        for u4 in range(4):
            for j in range(EMBED // LANES):
                sl = pl.ds(j * LANES, LANES)
                pad_v[r + u4, sl] = rows_v[r + u4, sl]
        return carry

    lax.fori_loop(0, BBLK // 4, cp_body, 0)

    for e8 in range(EG):
        for er in range(8):
            e = e8 * 8 + er
            colv = jnp.full((LANES,), e, jnp.int32)
            posv = jnp.full((LANES,), prow[e // LANES][e % LANES], jnp.float32)
            vals = [plsc.load_gather(pad_v, [rowvs[bb], colv])
                    for bb in range(8)]
            for bb in range(8):
                stag_v[e8, er, pl.ds(bb * LANES, LANES)] = vals[bb] + posv
        pltpu.async_copy(stag_v.at[e8], out_hbm.at[l, e8, b128], osem)


def _drain_out(out_hbm, stag_v, osem, u):
    l = u >> 5
    b128 = u & (NB - 1)
    for e8 in range(EG):
        pltpu.make_async_copy(stag_v.at[e8], out_hbm.at[l, e8, b128], osem).wait()


def _body(xt_hbm, tok_hbm, pos_hbm, out_hbm, idx_v, rows0, rows1, pad_v,
          stag0, stag1, pos_v, gsem0, gsem1, osem0, osem1):
    wid = lax.axis_index("s") * NC + lax.axis_index("c")
    ubase = wid * NUNIT
    pltpu.sync_copy(xt_hbm.at[pl.ds(ubase, NUNIT)], idx_v)
    pltpu.sync_copy(pos_hbm.at[pl.ds(0, L)], pos_v)

    pltpu.async_copy(tok_hbm.at[idx_v.at[0]], rows0, gsem0)

    def pair_body(i, carry):
        t = i * 2

        pltpu.async_copy(tok_hbm.at[idx_v.at[t + 1]], rows1, gsem1)
        pltpu.make_async_copy(tok_hbm.at[idx_v.at[t]], rows0, gsem0).wait()

        @pl.when(i > 0)
        def _():
            _drain_out(out_hbm, stag0, osem0, ubase + t - 2)

        _emit_unit(ubase + t, rows0, pad_v, pos_v, stag0, out_hbm, osem0)

        @pl.when(i < NUNIT // 2 - 1)
        def _():
            pltpu.async_copy(tok_hbm.at[idx_v.at[t + 2]], rows0, gsem0)

        pltpu.make_async_copy(tok_hbm.at[idx_v.at[t + 1]], rows1, gsem1).wait()

        @pl.when(i > 0)
        def _():
            _drain_out(out_hbm, stag1, osem1, ubase + t - 1)

        _emit_unit(ubase + t + 1, rows1, pad_v, pos_v, stag1, out_hbm, osem1)
        return carry

    lax.fori_loop(0, NUNIT // 2, pair_body, 0)
    _drain_out(out_hbm, stag0, osem0, ubase + NUNIT - 2)
    _drain_out(out_hbm, stag1, osem1, ubase + NUNIT - 1)


@jax.jit
def kernel(x, token_table, pos_table):
    xt = x.T.reshape(L * NB, BBLK)
    mesh = plsc.VectorSubcoreMesh(core_axis_name="c", subcore_axis_name="s")
    run = pl.kernel(
        _body,
        mesh=mesh,
        compiler_params=pltpu.CompilerParams(
            use_tc_tiling_on_sc=False, needs_layout_passes=False,
            disable_bounds_checks=True),
        out_type=jax.ShapeDtypeStruct((L, EG, NB, 8, BBLK), jnp.float32),
        scratch_types=[
            pltpu.VMEM((NUNIT, BBLK), jnp.int32),
            pltpu.VMEM((BBLK, EMBED), jnp.float32),
            pltpu.VMEM((BBLK, EMBED), jnp.float32),
            pltpu.VMEM((BBLK, ROWPAD), jnp.float32),
            pltpu.VMEM((EG, 8, BBLK), jnp.float32),
            pltpu.VMEM((EG, 8, BBLK), jnp.float32),
            pltpu.VMEM((L, EMBED), jnp.float32),
            pltpu.SemaphoreType.DMA,
            pltpu.SemaphoreType.DMA,
            pltpu.SemaphoreType.DMA,
            pltpu.SemaphoreType.DMA,
        ],
    )
    out5 = run(xt, token_table, pos_table)
    return out5.transpose(2, 4, 0, 1, 3).reshape(B, L, EMBED)


# speedup vs baseline: 1.2589x; 1.0059x over previous
import functools

import jax
import jax.numpy as jnp
from jax import lax
from jax.experimental import pallas as pl
from jax.experimental.pallas import tpu as pltpu, tpu_sc as plsc

EMBED = 64
B = 4096
L = 200

NC = 2
NS = 16
NW = NC * NS
BBLK = 128
NB = B // BBLK
NUNIT = (L * NB) // NW
EG = EMBED // 8
LANES = 16
ROWPAD = 65


def _emit_unit(u, rows_v, pad_v, pos_v, stag_v, out_hbm, osem):
    l = u >> 5
    b128 = u & (NB - 1)
    iot = lax.iota(jnp.int32, LANES)
    rowvs = [iot + bb * LANES for bb in range(8)]
    prow = [pos_v[l, pl.ds(j * LANES, LANES)] for j in range(EMBED // LANES)]

    def cp_body(k, carry):
        r = k * 4
        for u4 in range(4):
            for j in range(EMBED // LANES):
                sl = pl.ds(j * LANES, LANES)
                pad_v[r + u4, sl] = rows_v[r + u4, sl]
        return carry

    lax.fori_loop(0, BBLK // 4, cp_body, 0)

    for e8 in range(EG):
        for er in range(8):
            e = e8 * 8 + er
            colv = jnp.full((LANES,), e, jnp.int32)
            posv = jnp.full((LANES,), prow[e // LANES][e % LANES], jnp.float32)
            vals = [plsc.load_gather(pad_v, [rowvs[bb], colv])
                    for bb in range(8)]
            for bb in range(8):
                stag_v[e8, er, pl.ds(bb * LANES, LANES)] = vals[bb] + posv
        pltpu.async_copy(stag_v.at[e8], out_hbm.at[l, e8, b128], osem)


def _drain_out(out_hbm, stag_v, osem, u):
    l = u >> 5
    b128 = u & (NB - 1)
    for e8 in range(EG):
        pltpu.make_async_copy(stag_v.at[e8], out_hbm.at[l, e8, b128], osem).wait()


def _body(xt_hbm, tok_hbm, pos_hbm, out_hbm, idx_v, rows0, rows1, pad_v,
          stag0, stag1, pos_v, gsem0, gsem1, osem0, osem1):
    wid = lax.axis_index("s") * NC + lax.axis_index("c")
    ubase = wid * NUNIT
    pltpu.sync_copy(xt_hbm.at[pl.ds(ubase, NUNIT)], idx_v)
    pltpu.sync_copy(pos_hbm.at[pl.ds(0, L)], pos_v)

    pltpu.async_copy(tok_hbm.at[idx_v.at[0]], rows0, gsem0)

    def pair_body(i, carry):
        t = i * 2

        pltpu.async_copy(tok_hbm.at[idx_v.at[t + 1]], rows1, gsem1)
        pltpu.make_async_copy(tok_hbm.at[idx_v.at[t]], rows0, gsem0).wait()

        @pl.when(i > 0)
        def _():
            _drain_out(out_hbm, stag0, osem0, ubase + t - 2)

        _emit_unit(ubase + t, rows0, pad_v, pos_v, stag0, out_hbm, osem0)

        @pl.when(i < NUNIT // 2 - 1)
        def _():
            pltpu.async_copy(tok_hbm.at[idx_v.at[t + 2]], rows0, gsem0)

        pltpu.make_async_copy(tok_hbm.at[idx_v.at[t + 1]], rows1, gsem1).wait()

        @pl.when(i > 0)
        def _():
            _drain_out(out_hbm, stag1, osem1, ubase + t - 1)

        _emit_unit(ubase + t + 1, rows1, pad_v, pos_v, stag1, out_hbm, osem1)
        return carry

    lax.fori_loop(0, NUNIT // 2, pair_body, 0)
    _drain_out(out_hbm, stag0, osem0, ubase + NUNIT - 2)
    _drain_out(out_hbm, stag1, osem1, ubase + NUNIT - 1)


@jax.jit
def kernel(x, token_table, pos_table):
    xt = x.T.reshape(L * NB, BBLK)
    mesh = plsc.VectorSubcoreMesh(core_axis_name="c", subcore_axis_name="s")
    run = pl.kernel(
        _body,
        mesh=mesh,
        compiler_params=pltpu.CompilerParams(
            use_tc_tiling_on_sc=False, needs_layout_passes=False,
            disable_bounds_checks=True),
        out_type=jax.ShapeDtypeStruct((L, EG, NB, 8, BBLK), jnp.float32),
        scratch_types=[
            pltpu.VMEM((NUNIT, BBLK), jnp.int32),
            pltpu.VMEM((BBLK, EMBED), jnp.float32),
            pltpu.VMEM((BBLK, EMBED), jnp.float32),
            pltpu.VMEM((BBLK, ROWPAD), jnp.float32),
            pltpu.VMEM((EG, 8, BBLK), jnp.float32),
            pltpu.VMEM((EG, 8, BBLK), jnp.float32),
            pltpu.VMEM((L, EMBED), jnp.float32),
            pltpu.SemaphoreType.DMA,
            pltpu.SemaphoreType.DMA,
            pltpu.SemaphoreType.DMA,
            pltpu.SemaphoreType.DMA,
        ],
    )
    out5 = run(xt, token_table, pos_table)
    return out5.transpose(2, 4, 0, 1, 3).reshape(B, L, EMBED)


# speedup vs baseline: 1.2839x; 1.0199x over previous
import functools

import jax
import jax.numpy as jnp
from jax import lax
from jax.experimental import pallas as pl
from jax.experimental.pallas import tpu as pltpu, tpu_sc as plsc

EMBED = 64
B = 4096
L = 200

NC = 2
NS = 16
NW = NC * NS
CHUNK = 100
NCHUNK = (B * L) // (CHUNK * NW)
LANES = 16


def _add_pos_rows(rows_v, pos_v, poff):
    def row_body(k, rcarry):
        r = k * 2
        for u in range(2):
            for j in range(EMBED // LANES):
                sl = pl.ds(j * LANES, LANES)
                rows_v[r + u, sl] = rows_v[r + u, sl] + pos_v[poff + r + u, sl]
        return rcarry

    lax.fori_loop(0, CHUNK // 2, row_body, 0)


def _body(x_hbm, tok_hbm, pos_hbm, out_hbm, idx_v, rows0, rows1, pos_v,
          sem0, sem1):
    wid = lax.axis_index("s") * NC + lax.axis_index("c")
    base = wid * NCHUNK
    pltpu.sync_copy(x_hbm.at[pl.ds(base, NCHUNK)], idx_v)
    pltpu.sync_copy(pos_hbm.at[pl.ds(0, 2 * CHUNK)], pos_v)

    g0 = pltpu.async_copy(tok_hbm.at[idx_v.at[0]], rows0, sem0)

    def pair_body(i, carry):
        c = i * 2
        g1 = pltpu.async_copy(tok_hbm.at[idx_v.at[c + 1]], rows1, sem1)
        pltpu.make_async_copy(tok_hbm.at[idx_v.at[c]], rows0, sem0).wait()
        _add_pos_rows(rows0, pos_v, 0)
        pltpu.sync_copy(rows0, out_hbm.at[base + c])

        @pl.when(i < NCHUNK // 2 - 1)
        def _():
            pltpu.async_copy(tok_hbm.at[idx_v.at[c + 2]], rows0, sem0)

        pltpu.make_async_copy(tok_hbm.at[idx_v.at[c + 1]], rows1, sem1).wait()
        _add_pos_rows(rows1, pos_v, CHUNK)
        pltpu.sync_copy(rows1, out_hbm.at[base + c + 1])
        return carry

    lax.fori_loop(0, NCHUNK // 2, pair_body, 0)


@jax.jit
def kernel(x, token_table, pos_table):
    xf = x.reshape(NW * NCHUNK, CHUNK)
    mesh = plsc.VectorSubcoreMesh(core_axis_name="c", subcore_axis_name="s")
    run = pl.kernel(
        _body,
        mesh=mesh,
        compiler_params=pltpu.CompilerParams(use_tc_tiling_on_sc=False),
        out_type=jax.ShapeDtypeStruct((NW * NCHUNK, CHUNK, EMBED), jnp.float32),
        scratch_types=[
            pltpu.VMEM((NCHUNK, CHUNK), jnp.int32),
            pltpu.VMEM((CHUNK, EMBED), jnp.float32),
            pltpu.VMEM((CHUNK, EMBED), jnp.float32),
            pltpu.VMEM((2 * CHUNK, EMBED), jnp.float32),
            pltpu.SemaphoreType.DMA,
            pltpu.SemaphoreType.DMA,
        ],
    )
    out = run(xf, token_table, pos_table)
    return out.reshape(B, L, EMBED)
